# Initial kernel scaffold; baseline (speedup 1.0000x reference)
import jax
import jax.numpy as jnp
from jax.experimental import pallas as pl


def kernel(logits, labels):
    raise NotImplementedError("write your pallas kernel here")


# speedup vs baseline: 2.7110x; 2.7110x over previous
import functools

import jax
import jax.numpy as jnp
from jax import lax
from jax.experimental import pallas as pl
from jax.experimental.pallas import tpu as pltpu
from jax.experimental.pallas import tpu_sc as plsc

N_CLASSES = 19
N_BINS = 15
L = 16
NW = 32
NPIX_PER_IMG = 512 * 512
N_IMG = 4
N_PIX = N_IMG * NPIX_PER_IMG
CH = 2048
PER_W = NPIX_PER_IMG // NW
SUBCHUNKS = PER_W // CH
GROUPS = CH // L
NE = N_CLASSES * N_BINS
NEP = 288


def _body(logits_hbm, labels_hbm, out_hbm, buf, lab_buf, h_cnt, h_conf,
          h_hit, outbuf, sem):
    wid = lax.axis_index("s") * 2 + lax.axis_index("c")
    iota = lax.iota(jnp.int32, L)
    ones = jnp.full((L,), 1.0, jnp.float32)
    zeros = jnp.zeros((L,), jnp.float32)

    def zero_hist(j, _):
        h_cnt[pl.ds(j * L, L)] = zeros
        h_conf[pl.ds(j * L, L)] = zeros
        h_hit[pl.ds(j * L, L)] = zeros
        return 0

    lax.fori_loop(0, NEP, zero_hist, 0)

    def do_group(g, _):
        lab = lab_buf[pl.ds(g * L, L)]
        es = []
        s = None
        for i in range(N_CLASSES):
            x = buf[pl.ds(i * CH + g * L, L)]
            e = jnp.exp(x)
            es.append(e)
            s = e if s is None else s + e
        scale = 1.0 / s
        for i in range(N_CLASSES):
            conf = es[i] * scale
            b = (conf * jnp.float32(N_BINS)).astype(jnp.int32)
            b = jnp.minimum(b, N_BINS - 1)
            idx = b * L + (iota + i * (N_BINS * L))
            plsc.addupdate_scatter(h_cnt, [idx], ones)
            plsc.addupdate_scatter(h_conf, [idx], conf)
            plsc.addupdate_scatter(h_hit, [idx], ones, mask=lab == i)
        return 0

    for img in range(N_IMG):
        def do_chunk(sub, _, img=img):
            col = wid * PER_W + sub * CH
            copies = []
            for i in range(N_CLASSES):
                src = logits_hbm.at[
                    pl.ds((img * N_CLASSES + i) * NPIX_PER_IMG + col, CH)]
                copies.append(
                    pltpu.async_copy(src, buf.at[pl.ds(i * CH, CH)], sem))
            lcopy = pltpu.async_copy(
                labels_hbm.at[pl.ds(img * NPIX_PER_IMG + col, CH)],
                lab_buf, sem)
            for c in copies:
                c.wait()
            lcopy.wait()
            lax.fori_loop(0, GROUPS, do_group, 0)
            return 0

        lax.fori_loop(0, SUBCHUNKS, do_chunk, 0)

    for si, h in enumerate((h_cnt, h_conf, h_hit)):
        def reduce_block(eg, _, h=h, si=si):
            base = eg * L
            acc = jnp.zeros((L,), jnp.float32)
            for k in range(L):
                acc = acc + plsc.load_gather(h, [(base + iota) * L + k])
            outbuf[pl.ds(si * NEP + base, L)] = acc
            return 0

        lax.fori_loop(0, NEP // L, reduce_block, 0)

    pltpu.sync_copy(outbuf, out_hbm.at[wid])


@jax.jit
def _ece_hist(logits_flat, labels_flat):
    mesh = plsc.VectorSubcoreMesh(core_axis_name="c", subcore_axis_name="s")
    kern = pl.kernel(
        _body,
        out_type=jax.ShapeDtypeStruct((NW, 3 * NEP), jnp.float32),
        mesh=mesh,
        scratch_types=[
            pltpu.VMEM((N_CLASSES * CH,), jnp.float32),
            pltpu.VMEM((CH,), jnp.int32),
            pltpu.VMEM((NEP * L,), jnp.float32),
            pltpu.VMEM((NEP * L,), jnp.float32),
            pltpu.VMEM((NEP * L,), jnp.float32),
            pltpu.VMEM((3 * NEP,), jnp.float32),
            pltpu.SemaphoreType.DMA,
        ],
        compiler_params=pltpu.CompilerParams(needs_layout_passes=False),
    )
    return kern(logits_flat, labels_flat)


def kernel(logits, labels):
    logits_flat = logits.reshape(-1)
    labels_flat = labels.reshape(-1).astype(jnp.int32)
    partials = _ece_hist(logits_flat, labels_flat)
    sums = partials.sum(axis=0)
    count = sums[0:NE].reshape(N_CLASSES, N_BINS)
    confsum = sums[NEP:NEP + NE].reshape(N_CLASSES, N_BINS)
    hitsum = sums[2 * NEP:2 * NEP + NE].reshape(N_CLASSES, N_BINS)
    prop = count / float(N_PIX)
    safe = jnp.maximum(count, 1.0)
    acc = hitsum / safe
    avgconf = confsum / safe
    contrib = jnp.where(count > 0, jnp.abs(avgconf - acc) * prop, 0.0)
    sce = contrib.sum(axis=1).mean()
    return (sce, acc, avgconf, prop)
